# Initial kernel scaffold; baseline (speedup 1.0000x reference)
import jax
import jax.numpy as jnp
from jax.experimental import pallas as pl


def kernel(x, edge_index, W1, b1, W2, b2):
    raise NotImplementedError("write your pallas kernel here")


# speedup vs baseline: 76.6877x; 76.6877x over previous
import functools

import jax
import jax.numpy as jnp
from jax import lax
from jax.experimental import pallas as pl
from jax.experimental.pallas import tpu as pltpu
from jax.experimental.pallas import tpu_sc as plsc

N = 10000
E = 320000
D = 128
H = 32
L = 10

NUM_TILES = 16
CHUNK = E // NUM_TILES
NP = 10240
SLICE = NP // NUM_TILES
LANES = 16


def _rsqrt16(d):
    i = lax.bitcast_convert_type(d, jnp.int32)
    i = jnp.int32(0x5F3759DF) - lax.shift_right_logical(i, 1)
    y = lax.bitcast_convert_type(i, jnp.float32)
    for _ in range(3):
        y = y * (1.5 - 0.5 * d * y * y)
    return y


def _sc_body(src_hbm, dst_hbm, c_hbm, src_v, dst_v, hist_v, norm_v,
             tmp_v, slice_v, shared_h, shared_n):
    sid = lax.axis_index("s")
    cid = lax.axis_index("c")
    base_e = sid * CHUNK
    base_n = sid * SLICE
    zeros = jnp.zeros((LANES,), jnp.float32)
    ones = jnp.full((LANES,), 1.0, jnp.float32)
    lane_iota = lax.iota(jnp.int32, LANES)

    pltpu.sync_copy(src_hbm.at[pl.ds(base_e, CHUNK)], src_v)
    pltpu.sync_copy(dst_hbm.at[pl.ds(base_e, CHUNK)], dst_v)

    def zero_body(i, _):
        hist_v[pl.ds(i * LANES, LANES)] = zeros
        return 0
    lax.fori_loop(0, NP // LANES, zero_body, 0)

    def hist_body(i, _):
        idx = dst_v[pl.ds(i * LANES, LANES)]
        plsc.addupdate_scatter(hist_v, [idx], ones)
        return 0
    lax.fori_loop(0, CHUNK // LANES, hist_body, 0)

    pltpu.sync_copy(hist_v, shared_h.at[sid])
    plsc.subcore_barrier()
    for t in range(NUM_TILES):
        pltpu.sync_copy(shared_h.at[t, pl.ds(base_n, SLICE)], tmp_v.at[t])

    def norm_body(j, _):
        acc = tmp_v[0, pl.ds(j * LANES, LANES)]
        for t in range(1, NUM_TILES):
            acc = acc + tmp_v[t, pl.ds(j * LANES, LANES)]
        slice_v[pl.ds(j * LANES, LANES)] = _rsqrt16(acc + 1.0)
        return 0
    lax.fori_loop(0, SLICE // LANES, norm_body, 0)

    pltpu.sync_copy(slice_v, shared_n.at[pl.ds(base_n, SLICE)])
    plsc.subcore_barrier()
    pltpu.sync_copy(shared_n, norm_v)

    lax.fori_loop(0, NP // LANES, zero_body, 0)

    def seg_body(i, _):
        didx = dst_v[pl.ds(i * LANES, LANES)]
        sidx = src_v[pl.ds(i * LANES, LANES)]
        vals = plsc.load_gather(norm_v, [didx])
        plsc.addupdate_scatter(hist_v, [sidx], vals)
        return 0
    lax.fori_loop(0, CHUNK // LANES, seg_body, 0)

    pltpu.sync_copy(hist_v, shared_h.at[sid])
    plsc.subcore_barrier()
    for t in range(NUM_TILES):
        pltpu.sync_copy(shared_h.at[t, pl.ds(base_n, SLICE)], tmp_v.at[t])

    def c_body(j, _):
        acc = tmp_v[0, pl.ds(j * LANES, LANES)]
        for t in range(1, NUM_TILES):
            acc = acc + tmp_v[t, pl.ds(j * LANES, LANES)]
        n16 = norm_v[pl.ds(base_n + j * LANES, LANES)]
        c16 = n16 * acc + n16 * n16
        gidx = base_n + j * LANES + lane_iota
        slice_v[pl.ds(j * LANES, LANES)] = jnp.where(gidx < N, c16, 0.0)
        return 0
    lax.fori_loop(0, SLICE // LANES, c_body, 0)

    @pl.when(cid == 0)
    def _():
        pltpu.sync_copy(slice_v, c_hbm.at[pl.ds(base_n, SLICE)])


@jax.jit
def _sc_coeffs(src, dst):
    mesh = plsc.VectorSubcoreMesh(
        core_axis_name="c", subcore_axis_name="s", num_cores=1)
    return pl.kernel(
        _sc_body,
        out_type=jax.ShapeDtypeStruct((NP,), jnp.float32),
        mesh=mesh,
        compiler_params=pltpu.CompilerParams(needs_layout_passes=False),
        scratch_types=[
            pltpu.VMEM((CHUNK,), jnp.int32),
            pltpu.VMEM((CHUNK,), jnp.int32),
            pltpu.VMEM((NP,), jnp.float32),
            pltpu.VMEM((NP,), jnp.float32),
            pltpu.VMEM((NUM_TILES, SLICE), jnp.float32),
            pltpu.VMEM((SLICE,), jnp.float32),
            pltpu.VMEM_SHARED((NUM_TILES, NP), jnp.float32),
            pltpu.VMEM_SHARED((NP,), jnp.float32),
        ],
    )(src, dst)


def _tc_body(x_ref, c_ref, w1t_ref, b1_ref, w2t_ref, b2_ref, o_ref):
    hi = jax.lax.Precision.HIGHEST
    v = lax.dot_general(x_ref[...], c_ref[...], (((0,), (0,)), ((), ())),
                        precision=hi, preferred_element_type=jnp.float32)
    pooled = lax.dot_general(w1t_ref[...], v, (((1,), (0,)), ((), ())),
                             precision=hi, preferred_element_type=jnp.float32)
    pooled = pooled + jnp.float32(N) * b1_ref[...]
    logits = lax.dot_general(w2t_ref[...], pooled, (((1,), (0,)), ((), ())),
                             precision=hi, preferred_element_type=jnp.float32)
    logits = logits + b2_ref[...]
    m = jnp.max(logits, axis=0, keepdims=True)
    e = jnp.exp(logits - m)
    o_ref[...] = e / jnp.sum(e, axis=0, keepdims=True)


@jax.jit
def _tc_head(x, c_col, w1t, b1col, w2t, b2col):
    return pl.pallas_call(
        _tc_body,
        out_shape=jax.ShapeDtypeStruct((128, 1), jnp.float32),
    )(x, c_col, w1t, b1col, w2t, b2col)


def kernel(x, edge_index, W1, b1, W2, b2):
    src = edge_index[0]
    dst = edge_index[1]
    c = _sc_coeffs(src, dst)
    c_col = c[:N].reshape(N, 1)

    w1t = jnp.zeros((128, 128), jnp.float32).at[:H, :D].set(W1.T)
    b1col = jnp.zeros((128, 1), jnp.float32).at[:H, 0].set(b1)
    w2t = jnp.zeros((128, 128), jnp.float32).at[:L, :H].set(W2.T)
    b2col = jnp.full((128, 1), -1e30, jnp.float32).at[:L, 0].set(b2)

    out = _tc_head(x, c_col, w1t, b1col, w2t, b2col)
    return out[:L, 0]


# speedup vs baseline: 83.5571x; 1.0896x over previous
import jax
import jax.numpy as jnp
from jax import lax
from jax.experimental import pallas as pl
from jax.experimental.pallas import tpu as pltpu
from jax.experimental.pallas import tpu_sc as plsc

N = 10000
E = 320000
D = 128
H = 32
L = 10

NUM_TILES = 16
CHUNK = E // NUM_TILES
NP = 10240
SLICE = NP // NUM_TILES
TAIL = N - (NUM_TILES - 1) * SLICE
LANES = 16


def _rsqrt16(d):
    i = lax.bitcast_convert_type(d, jnp.int32)
    i = jnp.int32(0x5F3759DF) - lax.shift_right_logical(i, 1)
    y = lax.bitcast_convert_type(i, jnp.float32)
    for _ in range(3):
        y = y * (1.5 - 0.5 * d * y * y)
    return y


def _sc_body(src_hbm, dst_hbm, c_hbm, src_v, dst_v, hist_v, norm_v,
             tmp_v, slice_v, sem, shared_h, shared_n):
    sid = lax.axis_index("s")
    base_e = sid * CHUNK
    base_n = sid * SLICE
    zeros = jnp.zeros((LANES,), jnp.float32)
    ones = jnp.full((LANES,), 1.0, jnp.float32)

    pltpu.sync_copy(src_hbm.at[pl.ds(base_e, CHUNK)], src_v)
    pltpu.sync_copy(dst_hbm.at[pl.ds(base_e, CHUNK)], dst_v)

    def zero_body(i, _):
        hist_v[pl.ds(i * LANES, LANES)] = zeros
        return 0

    def stage_and_reduce():
        pltpu.sync_copy(hist_v, shared_h.at[sid])
        plsc.subcore_barrier()
        copies = [
            pltpu.async_copy(shared_h.at[t, pl.ds(base_n, SLICE)],
                             tmp_v.at[t], sem)
            for t in range(NUM_TILES)
        ]
        for cp in copies:
            cp.wait()

    lax.fori_loop(0, NP // LANES, zero_body, 0)

    def hist_body(i, _):
        idx = dst_v[pl.ds(i * LANES, LANES)]
        plsc.addupdate_scatter(hist_v, [idx], ones)
        return 0
    lax.fori_loop(0, CHUNK // LANES, hist_body, 0)

    stage_and_reduce()

    def norm_body(j, _):
        acc = tmp_v[0, pl.ds(j * LANES, LANES)]
        for t in range(1, NUM_TILES):
            acc = acc + tmp_v[t, pl.ds(j * LANES, LANES)]
        slice_v[pl.ds(j * LANES, LANES)] = _rsqrt16(acc + 1.0)
        return 0
    lax.fori_loop(0, SLICE // LANES, norm_body, 0)

    pltpu.sync_copy(slice_v, shared_n.at[pl.ds(base_n, SLICE)])
    plsc.subcore_barrier()
    pltpu.sync_copy(shared_n, norm_v)

    lax.fori_loop(0, NP // LANES, zero_body, 0)

    def seg_body(i, _):
        didx = dst_v[pl.ds(i * LANES, LANES)]
        sidx = src_v[pl.ds(i * LANES, LANES)]
        vals = plsc.load_gather(norm_v, [didx])
        plsc.addupdate_scatter(hist_v, [sidx], vals)
        return 0
    lax.fori_loop(0, CHUNK // LANES, seg_body, 0)

    stage_and_reduce()

    def c_body(j, _):
        acc = tmp_v[0, pl.ds(j * LANES, LANES)]
        for t in range(1, NUM_TILES):
            acc = acc + tmp_v[t, pl.ds(j * LANES, LANES)]
        n16 = norm_v[pl.ds(base_n + j * LANES, LANES)]
        slice_v[pl.ds(j * LANES, LANES)] = n16 * acc + n16 * n16
        return 0
    lax.fori_loop(0, SLICE // LANES, c_body, 0)

    @pl.when(sid < NUM_TILES - 1)
    def _():
        pltpu.sync_copy(slice_v, c_hbm.at[pl.ds(base_n, SLICE)])

    @pl.when(sid == NUM_TILES - 1)
    def _():
        pltpu.sync_copy(slice_v.at[pl.ds(0, TAIL)],
                        c_hbm.at[pl.ds(base_n, TAIL)])


@jax.jit
def _sc_coeffs(src, dst):
    mesh = plsc.VectorSubcoreMesh(
        core_axis_name="c", subcore_axis_name="s", num_cores=1)
    return pl.kernel(
        _sc_body,
        out_type=jax.ShapeDtypeStruct((N,), jnp.float32),
        mesh=mesh,
        compiler_params=pltpu.CompilerParams(needs_layout_passes=False),
        scratch_types=[
            pltpu.VMEM((CHUNK,), jnp.int32),
            pltpu.VMEM((CHUNK,), jnp.int32),
            pltpu.VMEM((NP,), jnp.float32),
            pltpu.VMEM((NP,), jnp.float32),
            pltpu.VMEM((NUM_TILES, SLICE), jnp.float32),
            pltpu.VMEM((SLICE,), jnp.float32),
            pltpu.SemaphoreType.DMA,
            pltpu.VMEM_SHARED((NUM_TILES, NP), jnp.float32),
            pltpu.VMEM_SHARED((NP,), jnp.float32),
        ],
    )(src, dst)


def _tc_body(x_ref, c_ref, w1_ref, b1_ref, w2_ref, b2_ref, o_ref):
    hi = jax.lax.Precision.HIGHEST
    v = lax.dot_general(c_ref[...], x_ref[...], (((0,), (0,)), ((), ())),
                        precision=hi, preferred_element_type=jnp.float32)
    pooled = lax.dot_general(v, w1_ref[...], (((1,), (0,)), ((), ())),
                             precision=hi, preferred_element_type=jnp.float32)
    pooled = pooled + jnp.float32(N) * b1_ref[...][None, :]
    logits = lax.dot_general(pooled, w2_ref[...], (((1,), (0,)), ((), ())),
                             precision=hi, preferred_element_type=jnp.float32)
    logits = logits + b2_ref[...][None, :]
    m = jnp.max(logits, axis=1, keepdims=True)
    e = jnp.exp(logits - m)
    p = e / jnp.sum(e, axis=1, keepdims=True)
    o_ref[...] = p[0]


@jax.jit
def _tc_head(x, c_col, w1, b1, w2, b2):
    return pl.pallas_call(
        _tc_body,
        out_shape=jax.ShapeDtypeStruct((L,), jnp.float32),
    )(x, c_col, w1, b1, w2, b2)


def kernel(x, edge_index, W1, b1, W2, b2):
    c = _sc_coeffs(edge_index[0], edge_index[1])
    return _tc_head(x, c.reshape(N, 1), W1, b1, W2, b2)


# speedup vs baseline: 90.5714x; 1.0839x over previous
import jax
import jax.numpy as jnp
from jax import lax
from jax.experimental import pallas as pl
from jax.experimental.pallas import tpu as pltpu
from jax.experimental.pallas import tpu_sc as plsc

N = 10000
E = 320000
D = 128
H = 32
L = 10

NUM_TILES = 16
CHUNK = E // NUM_TILES
NP = 10240
SLICE = NP // NUM_TILES
TAIL = N - (NUM_TILES - 1) * SLICE
LANES = 16


def _rsqrt16(d):
    i = lax.bitcast_convert_type(d, jnp.int32)
    i = jnp.int32(0x5F3759DF) - lax.shift_right_logical(i, 1)
    y = lax.bitcast_convert_type(i, jnp.float32)
    for _ in range(3):
        y = y * (1.5 - 0.5 * d * y * y)
    return y


def _sc_body(src_hbm, dst_hbm, c_hbm, src_v, dst_v, hist_v, norm_v,
             tmp_v, slice_v, sem, sem2, shared_h, shared_n):
    sid = lax.axis_index("s")
    base_e = sid * CHUNK
    base_n = sid * SLICE
    zeros = jnp.zeros((LANES,), jnp.float32)
    ones = jnp.full((LANES,), 1.0, jnp.float32)

    cp_src = pltpu.async_copy(src_hbm.at[pl.ds(base_e, CHUNK)], src_v, sem2)
    cp_dst = pltpu.async_copy(dst_hbm.at[pl.ds(base_e, CHUNK)], dst_v, sem)

    ZU = 8

    def zero_body(i, _):
        for u in range(ZU):
            hist_v[pl.ds((i * ZU + u) * LANES, LANES)] = zeros
        return 0

    def stage_and_reduce():
        pltpu.sync_copy(hist_v, shared_h.at[sid])
        plsc.subcore_barrier()
        copies = [
            pltpu.async_copy(shared_h.at[t, pl.ds(base_n, SLICE)],
                             tmp_v.at[t], sem)
            for t in range(NUM_TILES)
        ]
        for cp in copies:
            cp.wait()

    lax.fori_loop(0, NP // (LANES * ZU), zero_body, 0)
    cp_dst.wait()

    EU = 5

    def hist_body(i, _):
        for u in range(EU):
            idx = dst_v[pl.ds((i * EU + u) * LANES, LANES)]
            plsc.addupdate_scatter(hist_v, [idx], ones)
        return 0
    lax.fori_loop(0, CHUNK // (LANES * EU), hist_body, 0)

    stage_and_reduce()

    def norm_body(j, _):
        acc = tmp_v[0, pl.ds(j * LANES, LANES)]
        for t in range(1, NUM_TILES):
            acc = acc + tmp_v[t, pl.ds(j * LANES, LANES)]
        slice_v[pl.ds(j * LANES, LANES)] = _rsqrt16(acc + 1.0)
        return 0
    lax.fori_loop(0, SLICE // LANES, norm_body, 0)

    pltpu.sync_copy(slice_v, shared_n.at[pl.ds(base_n, SLICE)])
    plsc.subcore_barrier()
    pltpu.sync_copy(shared_n, norm_v)

    lax.fori_loop(0, NP // (LANES * ZU), zero_body, 0)
    cp_src.wait()

    def seg_body(i, _):
        for u in range(EU):
            didx = dst_v[pl.ds((i * EU + u) * LANES, LANES)]
            sidx = src_v[pl.ds((i * EU + u) * LANES, LANES)]
            vals = plsc.load_gather(norm_v, [didx])
            plsc.addupdate_scatter(hist_v, [sidx], vals)
        return 0
    lax.fori_loop(0, CHUNK // (LANES * EU), seg_body, 0)

    stage_and_reduce()

    def c_body(j, _):
        acc = tmp_v[0, pl.ds(j * LANES, LANES)]
        for t in range(1, NUM_TILES):
            acc = acc + tmp_v[t, pl.ds(j * LANES, LANES)]
        n16 = norm_v[pl.ds(base_n + j * LANES, LANES)]
        slice_v[pl.ds(j * LANES, LANES)] = n16 * acc + n16 * n16
        return 0
    lax.fori_loop(0, SLICE // LANES, c_body, 0)

    @pl.when(sid < NUM_TILES - 1)
    def _():
        pltpu.sync_copy(slice_v, c_hbm.at[pl.ds(base_n, SLICE)])

    @pl.when(sid == NUM_TILES - 1)
    def _():
        pltpu.sync_copy(slice_v.at[pl.ds(0, TAIL)],
                        c_hbm.at[pl.ds(base_n, TAIL)])


@jax.jit
def _sc_coeffs(src, dst):
    mesh = plsc.VectorSubcoreMesh(
        core_axis_name="c", subcore_axis_name="s", num_cores=1)
    return pl.kernel(
        _sc_body,
        out_type=jax.ShapeDtypeStruct((N,), jnp.float32),
        mesh=mesh,
        compiler_params=pltpu.CompilerParams(needs_layout_passes=False),
        scratch_types=[
            pltpu.VMEM((CHUNK,), jnp.int32),
            pltpu.VMEM((CHUNK,), jnp.int32),
            pltpu.VMEM((NP,), jnp.float32),
            pltpu.VMEM((NP,), jnp.float32),
            pltpu.VMEM((NUM_TILES, SLICE), jnp.float32),
            pltpu.VMEM((SLICE,), jnp.float32),
            pltpu.SemaphoreType.DMA,
            pltpu.SemaphoreType.DMA,
            pltpu.VMEM_SHARED((NUM_TILES, NP), jnp.float32),
            pltpu.VMEM_SHARED((NP,), jnp.float32),
        ],
    )(src, dst)


def _tc_body(x_ref, c_ref, w1_ref, b1_ref, w2_ref, b2_ref, o_ref):
    hi = jax.lax.Precision.HIGHEST
    v = lax.dot_general(c_ref[...], x_ref[...], (((0,), (0,)), ((), ())),
                        precision=hi, preferred_element_type=jnp.float32)
    pooled = lax.dot_general(v, w1_ref[...], (((1,), (0,)), ((), ())),
                             precision=hi, preferred_element_type=jnp.float32)
    pooled = pooled + jnp.float32(N) * b1_ref[...][None, :]
    logits = lax.dot_general(pooled, w2_ref[...], (((1,), (0,)), ((), ())),
                             precision=hi, preferred_element_type=jnp.float32)
    logits = logits + b2_ref[...][None, :]
    m = jnp.max(logits, axis=1, keepdims=True)
    e = jnp.exp(logits - m)
    p = e / jnp.sum(e, axis=1, keepdims=True)
    o_ref[...] = p[0]


@jax.jit
def _tc_head(x, c_col, w1, b1, w2, b2):
    return pl.pallas_call(
        _tc_body,
        out_shape=jax.ShapeDtypeStruct((L,), jnp.float32),
    )(x, c_col, w1, b1, w2, b2)


def kernel(x, edge_index, W1, b1, W2, b2):
    c = _sc_coeffs(edge_index[0], edge_index[1])
    return _tc_head(x, c.reshape(N, 1), W1, b1, W2, b2)


# speedup vs baseline: 107.9169x; 1.1915x over previous
import jax
import jax.numpy as jnp
from jax import lax
from jax.experimental import pallas as pl
from jax.experimental.pallas import tpu as pltpu
from jax.experimental.pallas import tpu_sc as plsc

N = 10000
E = 320000
D = 128
H = 32
L = 10

NUM_TILES = 16
NUM_CORES = 2
CHUNK = E // NUM_TILES
HALF = CHUNK // NUM_CORES
NP = 10240
SLICE = NP // NUM_TILES
LANES = 16
ZU = 8
EU = 5


def _rsqrt16(d):
    i = lax.bitcast_convert_type(d, jnp.int32)
    i = jnp.int32(0x5F3759DF) - lax.shift_right_logical(i, 1)
    y = lax.bitcast_convert_type(i, jnp.float32)
    for _ in range(3):
        y = y * (1.5 - 0.5 * d * y * y)
    return y


def _sc_body(src_hbm, dst_hbm, s0_hbm, s1_hbm, norm_hbm,
             src_v, dst_v, hist_v, norm_v, tmp_v, slice_v, sem, sem2,
             shared_h, shared_n):
    sid = lax.axis_index("s")
    cid = lax.axis_index("c")
    base_e = sid * CHUNK
    base_n = sid * SLICE
    zeros = jnp.zeros((LANES,), jnp.float32)
    ones = jnp.full((LANES,), 1.0, jnp.float32)

    cp_src = pltpu.async_copy(
        src_hbm.at[pl.ds(base_e + cid * HALF, HALF)], src_v, sem2)
    cp_dst = pltpu.async_copy(dst_hbm.at[pl.ds(base_e, CHUNK)], dst_v, sem)

    def zero_body(i, _):
        for u in range(ZU):
            hist_v[pl.ds((i * ZU + u) * LANES, LANES)] = zeros
        return 0

    def stage_and_reduce():
        pltpu.sync_copy(hist_v, shared_h.at[sid])
        plsc.subcore_barrier()
        copies = [
            pltpu.async_copy(shared_h.at[t, pl.ds(base_n, SLICE)],
                             tmp_v.at[t], sem)
            for t in range(NUM_TILES)
        ]
        for cp in copies:
            cp.wait()

    lax.fori_loop(0, NP // (LANES * ZU), zero_body, 0)
    cp_dst.wait()

    def hist_body(i, _):
        for u in range(EU):
            idx = dst_v[pl.ds((i * EU + u) * LANES, LANES)]
            plsc.addupdate_scatter(hist_v, [idx], ones)
        return 0
    lax.fori_loop(0, CHUNK // (LANES * EU), hist_body, 0)

    stage_and_reduce()

    def norm_body(j, _):
        acc = tmp_v[0, pl.ds(j * LANES, LANES)]
        for t in range(1, NUM_TILES):
            acc = acc + tmp_v[t, pl.ds(j * LANES, LANES)]
        slice_v[pl.ds(j * LANES, LANES)] = _rsqrt16(acc + 1.0)
        return 0
    lax.fori_loop(0, SLICE // LANES, norm_body, 0)

    pltpu.sync_copy(slice_v, shared_n.at[pl.ds(base_n, SLICE)])
    plsc.subcore_barrier()
    pltpu.sync_copy(shared_n, norm_v)

    @pl.when(cid == 0)
    def _():
        pltpu.sync_copy(slice_v, norm_hbm.at[pl.ds(base_n, SLICE)])

    lax.fori_loop(0, NP // (LANES * ZU), zero_body, 0)
    cp_src.wait()
    dst_off = cid * HALF

    def seg_body(i, _):
        for u in range(EU):
            off = (i * EU + u) * LANES
            didx = dst_v[pl.ds(dst_off + off, LANES)]
            sidx = src_v[pl.ds(off, LANES)]
            vals = plsc.load_gather(norm_v, [didx])
            plsc.addupdate_scatter(hist_v, [sidx], vals)
        return 0
    lax.fori_loop(0, HALF // (LANES * EU), seg_body, 0)

    stage_and_reduce()

    def s_body(j, _):
        acc = tmp_v[0, pl.ds(j * LANES, LANES)]
        for t in range(1, NUM_TILES):
            acc = acc + tmp_v[t, pl.ds(j * LANES, LANES)]
        slice_v[pl.ds(j * LANES, LANES)] = acc
        return 0
    lax.fori_loop(0, SLICE // LANES, s_body, 0)

    @pl.when(cid == 0)
    def _():
        pltpu.sync_copy(slice_v, s0_hbm.at[pl.ds(base_n, SLICE)])

    @pl.when(cid == 1)
    def _():
        pltpu.sync_copy(slice_v, s1_hbm.at[pl.ds(base_n, SLICE)])


@jax.jit
def _sc_coeffs(src, dst):
    mesh = plsc.VectorSubcoreMesh(
        core_axis_name="c", subcore_axis_name="s", num_cores=NUM_CORES)
    return pl.kernel(
        _sc_body,
        out_type=(
            jax.ShapeDtypeStruct((NP,), jnp.float32),
            jax.ShapeDtypeStruct((NP,), jnp.float32),
            jax.ShapeDtypeStruct((NP,), jnp.float32),
        ),
        mesh=mesh,
        compiler_params=pltpu.CompilerParams(needs_layout_passes=False),
        scratch_types=[
            pltpu.VMEM((HALF,), jnp.int32),
            pltpu.VMEM((CHUNK,), jnp.int32),
            pltpu.VMEM((NP,), jnp.float32),
            pltpu.VMEM((NP,), jnp.float32),
            pltpu.VMEM((NUM_TILES, SLICE), jnp.float32),
            pltpu.VMEM((SLICE,), jnp.float32),
            pltpu.SemaphoreType.DMA,
            pltpu.SemaphoreType.DMA,
            pltpu.VMEM_SHARED((NUM_TILES, NP), jnp.float32),
            pltpu.VMEM_SHARED((NP,), jnp.float32),
        ],
    )(src, dst)


def _tc_body(x_ref, s0_ref, s1_ref, nrm_ref, w1_ref, b1_ref, w2_ref, b2_ref,
             o_ref):
    hi = jax.lax.Precision.HIGHEST
    nrm = nrm_ref[...]
    c = nrm * (s0_ref[...] + s1_ref[...]) + nrm * nrm
    c_row = c[:N].reshape(1, N)
    v = lax.dot_general(c_row, x_ref[...], (((1,), (0,)), ((), ())),
                        precision=hi, preferred_element_type=jnp.float32)
    pooled = lax.dot_general(v, w1_ref[...], (((1,), (0,)), ((), ())),
                             precision=hi, preferred_element_type=jnp.float32)
    pooled = pooled + jnp.float32(N) * b1_ref[...][None, :]
    logits = lax.dot_general(pooled, w2_ref[...], (((1,), (0,)), ((), ())),
                             precision=hi, preferred_element_type=jnp.float32)
    logits = logits + b2_ref[...][None, :]
    m = jnp.max(logits, axis=1, keepdims=True)
    e = jnp.exp(logits - m)
    p = e / jnp.sum(e, axis=1, keepdims=True)
    o_ref[...] = p[0]


@jax.jit
def _tc_head(x, s0, s1, nrm, w1, b1, w2, b2):
    return pl.pallas_call(
        _tc_body,
        out_shape=jax.ShapeDtypeStruct((L,), jnp.float32),
    )(x, s0, s1, nrm, w1, b1, w2, b2)


def kernel(x, edge_index, W1, b1, W2, b2):
    s0, s1, nrm = _sc_coeffs(edge_index[0], edge_index[1])
    return _tc_head(x, s0, s1, nrm, W1, b1, W2, b2)


# speedup vs baseline: 131.7583x; 1.2209x over previous
import jax
import jax.numpy as jnp
from jax import lax
from jax.experimental import pallas as pl
from jax.experimental.pallas import tpu as pltpu
from jax.experimental.pallas import tpu_sc as plsc

N = 10000
E = 320000
D = 128
H = 32
L = 10

NUM_TILES = 16
NUM_CORES = 2
CHUNK = E // NUM_TILES
HALF = CHUNK // NUM_CORES
NP = 10240
SLICE = NP // NUM_TILES
LANES = 16
ZU = 8
EU = 5
WIN = CHUNK + 96


def _rsqrt16(d):
    i = lax.bitcast_convert_type(d, jnp.int32)
    i = jnp.int32(0x5F3759DF) - lax.shift_right_logical(i, 1)
    y = lax.bitcast_convert_type(i, jnp.float32)
    for _ in range(3):
        y = y * (1.5 - 0.5 * d * y * y)
    return y


def _sc_body(edge_hbm, s0_hbm, s1_hbm, norm_hbm,
             edge_v, hist_v, norm_v, tmp_v, slice_v, sem,
             shared_h, shared_n):
    sid = lax.axis_index("s")
    cid = lax.axis_index("c")
    base_e = sid * CHUNK
    base_n = sid * SLICE
    zeros = jnp.zeros((LANES,), jnp.float32)
    ones = jnp.full((LANES,), 1.0, jnp.float32)

    start_a = base_e - lax.rem(base_e, 128)
    start_a = pl.multiple_of(start_a, 128)
    off = base_e - start_a
    cp_edge = pltpu.async_copy(
        edge_hbm.at[:, pl.ds(start_a, WIN)], edge_v, sem)

    def zero_body(i, _):
        for u in range(ZU):
            hist_v[pl.ds((i * ZU + u) * LANES, LANES)] = zeros
        return 0

    def stage_and_reduce():
        pltpu.sync_copy(hist_v, shared_h.at[sid])
        plsc.subcore_barrier()
        copies = [
            pltpu.async_copy(shared_h.at[t, pl.ds(base_n, SLICE)],
                             tmp_v.at[t], sem)
            for t in range(NUM_TILES)
        ]
        for cp in copies:
            cp.wait()

    lax.fori_loop(0, NP // (LANES * ZU), zero_body, 0)
    cp_edge.wait()

    def hist_body(i, _):
        for u in range(EU):
            idx = edge_v[1, pl.ds(off + (i * EU + u) * LANES, LANES)]
            plsc.addupdate_scatter(hist_v, [idx], ones)
        return 0
    lax.fori_loop(0, CHUNK // (LANES * EU), hist_body, 0)

    stage_and_reduce()

    def norm_body(j, _):
        acc = tmp_v[0, pl.ds(j * LANES, LANES)]
        for t in range(1, NUM_TILES):
            acc = acc + tmp_v[t, pl.ds(j * LANES, LANES)]
        slice_v[pl.ds(j * LANES, LANES)] = _rsqrt16(acc + 1.0)
        return 0
    lax.fori_loop(0, SLICE // LANES, norm_body, 0)

    pltpu.sync_copy(slice_v, shared_n.at[pl.ds(base_n, SLICE)])
    plsc.subcore_barrier()
    pltpu.sync_copy(shared_n, norm_v)

    @pl.when(cid == 0)
    def _():
        pltpu.sync_copy(slice_v, norm_hbm.at[pl.ds(base_n, SLICE)])

    lax.fori_loop(0, NP // (LANES * ZU), zero_body, 0)
    half_off = off + cid * HALF

    def seg_body(i, _):
        for u in range(EU):
            o = half_off + (i * EU + u) * LANES
            didx = edge_v[1, pl.ds(o, LANES)]
            sidx = edge_v[0, pl.ds(o, LANES)]
            vals = plsc.load_gather(norm_v, [didx])
            plsc.addupdate_scatter(hist_v, [sidx], vals)
        return 0
    lax.fori_loop(0, HALF // (LANES * EU), seg_body, 0)

    stage_and_reduce()

    def s_body(j, _):
        acc = tmp_v[0, pl.ds(j * LANES, LANES)]
        for t in range(1, NUM_TILES):
            acc = acc + tmp_v[t, pl.ds(j * LANES, LANES)]
        slice_v[pl.ds(j * LANES, LANES)] = acc
        return 0
    lax.fori_loop(0, SLICE // LANES, s_body, 0)

    @pl.when(cid == 0)
    def _():
        pltpu.sync_copy(slice_v, s0_hbm.at[pl.ds(base_n, SLICE)])

    @pl.when(cid == 1)
    def _():
        pltpu.sync_copy(slice_v, s1_hbm.at[pl.ds(base_n, SLICE)])


@jax.jit
def _sc_coeffs(edge_index):
    mesh = plsc.VectorSubcoreMesh(
        core_axis_name="c", subcore_axis_name="s", num_cores=NUM_CORES)
    return pl.kernel(
        _sc_body,
        out_type=(
            jax.ShapeDtypeStruct((NP,), jnp.float32),
            jax.ShapeDtypeStruct((NP,), jnp.float32),
            jax.ShapeDtypeStruct((NP,), jnp.float32),
        ),
        mesh=mesh,
        compiler_params=pltpu.CompilerParams(needs_layout_passes=False),
        scratch_types=[
            pltpu.VMEM((2, WIN), jnp.int32),
            pltpu.VMEM((NP,), jnp.float32),
            pltpu.VMEM((NP,), jnp.float32),
            pltpu.VMEM((NUM_TILES, SLICE), jnp.float32),
            pltpu.VMEM((SLICE,), jnp.float32),
            pltpu.SemaphoreType.DMA,
            pltpu.VMEM_SHARED((NUM_TILES, NP), jnp.float32),
            pltpu.VMEM_SHARED((NP,), jnp.float32),
        ],
    )(edge_index)


def _tc_body(x_ref, s0_ref, s1_ref, nrm_ref, w1_ref, b1_ref, w2_ref, b2_ref,
             o_ref):
    hi = jax.lax.Precision.HIGHEST
    nrm = nrm_ref[...]
    c = nrm * (s0_ref[...] + s1_ref[...]) + nrm * nrm
    c_row = c[:N].reshape(1, N)
    v = lax.dot_general(c_row, x_ref[...], (((1,), (0,)), ((), ())),
                        precision=hi, preferred_element_type=jnp.float32)
    pooled = lax.dot_general(v, w1_ref[...], (((1,), (0,)), ((), ())),
                             precision=hi, preferred_element_type=jnp.float32)
    pooled = pooled + jnp.float32(N) * b1_ref[...][None, :]
    logits = lax.dot_general(pooled, w2_ref[...], (((1,), (0,)), ((), ())),
                             precision=hi, preferred_element_type=jnp.float32)
    logits = logits + b2_ref[...][None, :]
    m = jnp.max(logits, axis=1, keepdims=True)
    e = jnp.exp(logits - m)
    p = e / jnp.sum(e, axis=1, keepdims=True)
    o_ref[...] = p[0]


@jax.jit
def _tc_head(x, s0, s1, nrm, w1, b1, w2, b2):
    return pl.pallas_call(
        _tc_body,
        out_shape=jax.ShapeDtypeStruct((L,), jnp.float32),
    )(x, s0, s1, nrm, w1, b1, w2, b2)


def kernel(x, edge_index, W1, b1, W2, b2):
    s0, s1, nrm = _sc_coeffs(edge_index)
    return _tc_head(x, s0, s1, nrm, W1, b1, W2, b2)


# speedup vs baseline: 167.3600x; 1.2702x over previous
import jax
import jax.numpy as jnp
from jax import lax
from jax.experimental import pallas as pl
from jax.experimental.pallas import tpu as pltpu
from jax.experimental.pallas import tpu_sc as plsc

N = 10000
E = 320000
D = 128
H = 32
L = 10

NUM_TILES = 16
NUM_CORES = 2
CHUNK = E // NUM_TILES
HALF = CHUNK // NUM_CORES
NP = 10240
SLICE = NP // NUM_TILES
LANES = 16
ZU = 8
EU = 5
WIN = CHUNK + 96


def _rsqrt16(d):
    i = lax.bitcast_convert_type(d, jnp.int32)
    i = jnp.int32(0x5F3759DF) - lax.shift_right_logical(i, 1)
    y = lax.bitcast_convert_type(i, jnp.float32)
    for _ in range(3):
        y = y * (1.5 - 0.5 * d * y * y)
    return y


def _sc_body(edge_hbm, s0_hbm, s1_hbm, norm_hbm,
             edge_v, hist_v, norm_v, tmp_v, slice_v, sem,
             shared_h, shared_n):
    sid = lax.axis_index("s")
    cid = lax.axis_index("c")
    base_e = sid * CHUNK
    base_n = sid * SLICE
    zeros = jnp.zeros((LANES,), jnp.float32)
    ones = jnp.full((LANES,), 1.0, jnp.float32)

    start_a = base_e - lax.rem(base_e, 128)
    start_a = pl.multiple_of(start_a, 128)
    off = base_e - start_a
    cp_edge = pltpu.async_copy(
        edge_hbm.at[:, pl.ds(start_a, WIN)], edge_v, sem)

    def zero_body(i, _):
        for u in range(ZU):
            hist_v[pl.ds((i * ZU + u) * LANES, LANES)] = zeros
        return 0

    def stage_and_reduce():
        pltpu.sync_copy(hist_v, shared_h.at[sid])
        plsc.subcore_barrier()
        pltpu.sync_copy(shared_h.at[:, pl.ds(base_n, SLICE)], tmp_v)

    lax.fori_loop(0, NP // (LANES * ZU), zero_body, 0)
    cp_edge.wait()

    def hist_body(i, carry):
        a, b = carry
        for u in range(EU):
            g = i * EU + u
            o_pf = off + jnp.minimum((g + 2) * LANES, CHUNK - LANES)
            nxt = edge_v[1, pl.ds(o_pf, LANES)]
            plsc.addupdate_scatter(hist_v, [a], ones)
            a, b = b, nxt
        return (a, b)
    lax.fori_loop(0, CHUNK // (LANES * EU), hist_body,
                  (edge_v[1, pl.ds(off, LANES)],
                   edge_v[1, pl.ds(off + LANES, LANES)]))

    stage_and_reduce()

    def norm_body(j, _):
        acc = tmp_v[0, pl.ds(j * LANES, LANES)]
        for t in range(1, NUM_TILES):
            acc = acc + tmp_v[t, pl.ds(j * LANES, LANES)]
        slice_v[pl.ds(j * LANES, LANES)] = _rsqrt16(acc + 1.0)
        return 0
    lax.fori_loop(0, SLICE // LANES, norm_body, 0)

    pltpu.sync_copy(slice_v, shared_n.at[pl.ds(base_n, SLICE)])
    plsc.subcore_barrier()
    pltpu.sync_copy(shared_n, norm_v)

    @pl.when(cid == 0)
    def _():
        pltpu.sync_copy(slice_v, norm_hbm.at[pl.ds(base_n, SLICE)])

    lax.fori_loop(0, NP // (LANES * ZU), zero_body, 0)
    half_off = off + cid * HALF

    def seg_body(i, carry):
        sa, va, sb, db = carry
        for u in range(EU):
            g = i * EU + u
            o_pf = half_off + jnp.minimum((g + 2) * LANES, HALF - LANES)
            sn = edge_v[0, pl.ds(o_pf, LANES)]
            dn = edge_v[1, pl.ds(o_pf, LANES)]
            vb = plsc.load_gather(norm_v, [db])
            plsc.addupdate_scatter(hist_v, [sa], va)
            sa, va, sb, db = sb, vb, sn, dn
        return (sa, va, sb, db)
    d0 = edge_v[1, pl.ds(half_off, LANES)]
    lax.fori_loop(0, HALF // (LANES * EU), seg_body,
                  (edge_v[0, pl.ds(half_off, LANES)],
                   plsc.load_gather(norm_v, [d0]),
                   edge_v[0, pl.ds(half_off + LANES, LANES)],
                   edge_v[1, pl.ds(half_off + LANES, LANES)]))

    stage_and_reduce()

    def s_body(j, _):
        acc = tmp_v[0, pl.ds(j * LANES, LANES)]
        for t in range(1, NUM_TILES):
            acc = acc + tmp_v[t, pl.ds(j * LANES, LANES)]
        slice_v[pl.ds(j * LANES, LANES)] = acc
        return 0
    lax.fori_loop(0, SLICE // LANES, s_body, 0)

    @pl.when(cid == 0)
    def _():
        pltpu.sync_copy(slice_v, s0_hbm.at[pl.ds(base_n, SLICE)])

    @pl.when(cid == 1)
    def _():
        pltpu.sync_copy(slice_v, s1_hbm.at[pl.ds(base_n, SLICE)])


@jax.jit
def _sc_coeffs(edge_index):
    mesh = plsc.VectorSubcoreMesh(
        core_axis_name="c", subcore_axis_name="s", num_cores=NUM_CORES)
    return pl.kernel(
        _sc_body,
        out_type=(
            jax.ShapeDtypeStruct((NP,), jnp.float32),
            jax.ShapeDtypeStruct((NP,), jnp.float32),
            jax.ShapeDtypeStruct((NP,), jnp.float32),
        ),
        mesh=mesh,
        compiler_params=pltpu.CompilerParams(needs_layout_passes=False),
        scratch_types=[
            pltpu.VMEM((2, WIN), jnp.int32),
            pltpu.VMEM((NP,), jnp.float32),
            pltpu.VMEM((NP,), jnp.float32),
            pltpu.VMEM((NUM_TILES, SLICE), jnp.float32),
            pltpu.VMEM((SLICE,), jnp.float32),
            pltpu.SemaphoreType.DMA,
            pltpu.VMEM_SHARED((NUM_TILES, NP), jnp.float32),
            pltpu.VMEM_SHARED((NP,), jnp.float32),
        ],
    )(edge_index)


def _tc_body(x_ref, s0_ref, s1_ref, nrm_ref, w1_ref, b1_ref, w2_ref, b2_ref,
             o_ref):
    hi = jax.lax.Precision.HIGHEST
    nrm = nrm_ref[...]
    c = nrm * (s0_ref[...] + s1_ref[...]) + nrm * nrm
    c_row = c[:N].reshape(1, N)
    v = lax.dot_general(c_row, x_ref[...], (((1,), (0,)), ((), ())),
                        precision=hi, preferred_element_type=jnp.float32)
    pooled = lax.dot_general(v, w1_ref[...], (((1,), (0,)), ((), ())),
                             precision=hi, preferred_element_type=jnp.float32)
    pooled = pooled + jnp.float32(N) * b1_ref[...][None, :]
    logits = lax.dot_general(pooled, w2_ref[...], (((1,), (0,)), ((), ())),
                             precision=hi, preferred_element_type=jnp.float32)
    logits = logits + b2_ref[...][None, :]
    m = jnp.max(logits, axis=1, keepdims=True)
    e = jnp.exp(logits - m)
    p = e / jnp.sum(e, axis=1, keepdims=True)
    o_ref[...] = p[0]


@jax.jit
def _tc_head(x, s0, s1, nrm, w1, b1, w2, b2):
    return pl.pallas_call(
        _tc_body,
        out_shape=jax.ShapeDtypeStruct((L,), jnp.float32),
    )(x, s0, s1, nrm, w1, b1, w2, b2)


def kernel(x, edge_index, W1, b1, W2, b2):
    s0, s1, nrm = _sc_coeffs(edge_index)
    return _tc_head(x, s0, s1, nrm, W1, b1, W2, b2)


# speedup vs baseline: 169.3374x; 1.0118x over previous
import jax
import jax.numpy as jnp
from jax import lax
from jax.experimental import pallas as pl
from jax.experimental.pallas import tpu as pltpu
from jax.experimental.pallas import tpu_sc as plsc

N = 10000
E = 320000
D = 128
H = 32
L = 10

NUM_TILES = 16
NUM_CORES = 2
CHUNK = E // NUM_TILES
HALF = CHUNK // NUM_CORES
NP = 10240
SLICE = NP // NUM_TILES
LANES = 16
ZU = 8
EU = 5
WIN = CHUNK + 96
W1 = 10112
NGRP = CHUNK // LANES
G1 = 620


def _rsqrt16(d):
    i = lax.bitcast_convert_type(d, jnp.int32)
    i = jnp.int32(0x5F3759DF) - lax.shift_right_logical(i, 1)
    y = lax.bitcast_convert_type(i, jnp.float32)
    for _ in range(3):
        y = y * (1.5 - 0.5 * d * y * y)
    return y


def _sc_body(edge_hbm, s0_hbm, s1_hbm, norm_hbm,
             edge_v, hist_v, norm_v, tmp_v, slice_v, sem, sem2,
             shared_h, shared_n):
    sid = lax.axis_index("s")
    cid = lax.axis_index("c")
    base_e = sid * CHUNK
    base_n = sid * SLICE
    zeros = jnp.zeros((LANES,), jnp.float32)
    ones = jnp.full((LANES,), 1.0, jnp.float32)

    start_a = base_e - lax.rem(base_e, 128)
    start_a = pl.multiple_of(start_a, 128)
    off = base_e - start_a
    cp_e1 = pltpu.async_copy(
        edge_hbm.at[:, pl.ds(start_a, W1)], edge_v.at[:, pl.ds(0, W1)], sem)
    cp_e2 = pltpu.async_copy(
        edge_hbm.at[:, pl.ds(start_a + W1, WIN - W1)],
        edge_v.at[:, pl.ds(W1, WIN - W1)], sem2)

    def zero_body(i, _):
        for u in range(ZU):
            hist_v[pl.ds((i * ZU + u) * LANES, LANES)] = zeros
        return 0

    def stage_and_reduce():
        pltpu.sync_copy(hist_v, shared_h.at[sid])
        plsc.subcore_barrier()
        pltpu.sync_copy(shared_h.at[:, pl.ds(base_n, SLICE)], tmp_v)

    lax.fori_loop(0, NP // (LANES * ZU), zero_body, 0)
    cp_e1.wait()

    def make_hist_body(g0, clamp):
        def body(i, carry):
            a, b = carry
            for u in range(EU):
                g = g0 + i * EU + u
                o_pf = off + jnp.minimum((g + 2) * LANES, clamp)
                nxt = edge_v[1, pl.ds(o_pf, LANES)]
                plsc.addupdate_scatter(hist_v, [a], ones)
                a, b = b, nxt
            return (a, b)
        return body

    lax.fori_loop(0, G1 // EU, make_hist_body(0, (G1 - 1) * LANES),
                  (edge_v[1, pl.ds(off, LANES)],
                   edge_v[1, pl.ds(off + LANES, LANES)]))
    cp_e2.wait()
    lax.fori_loop(0, (NGRP - G1) // EU,
                  make_hist_body(G1, CHUNK - LANES),
                  (edge_v[1, pl.ds(off + G1 * LANES, LANES)],
                   edge_v[1, pl.ds(off + (G1 + 1) * LANES, LANES)]))

    stage_and_reduce()

    def tree_sum(j):
        vals = [tmp_v[t, pl.ds(j * LANES, LANES)] for t in range(NUM_TILES)]
        while len(vals) > 1:
            vals = [vals[k] + vals[k + 1] for k in range(0, len(vals), 2)]
        return vals[0]

    def norm_body(j, _):
        slice_v[pl.ds(j * LANES, LANES)] = _rsqrt16(tree_sum(j) + 1.0)
        return 0
    lax.fori_loop(0, SLICE // LANES, norm_body, 0)

    pltpu.sync_copy(slice_v, shared_n.at[pl.ds(base_n, SLICE)])
    plsc.subcore_barrier()
    pltpu.sync_copy(shared_n, norm_v)

    @pl.when(cid == 0)
    def _():
        pltpu.sync_copy(slice_v, norm_hbm.at[pl.ds(base_n, SLICE)])

    lax.fori_loop(0, NP // (LANES * ZU), zero_body, 0)
    half_off = off + cid * HALF

    def seg_body(i, carry):
        sa, va, sb, db = carry
        for u in range(EU):
            g = i * EU + u
            o_pf = half_off + jnp.minimum((g + 2) * LANES, HALF - LANES)
            sn = edge_v[0, pl.ds(o_pf, LANES)]
            dn = edge_v[1, pl.ds(o_pf, LANES)]
            vb = plsc.load_gather(norm_v, [db])
            plsc.addupdate_scatter(hist_v, [sa], va)
            sa, va, sb, db = sb, vb, sn, dn
        return (sa, va, sb, db)
    d0 = edge_v[1, pl.ds(half_off, LANES)]
    lax.fori_loop(0, HALF // (LANES * EU), seg_body,
                  (edge_v[0, pl.ds(half_off, LANES)],
                   plsc.load_gather(norm_v, [d0]),
                   edge_v[0, pl.ds(half_off + LANES, LANES)],
                   edge_v[1, pl.ds(half_off + LANES, LANES)]))

    stage_and_reduce()

    def s_body(j, _):
        slice_v[pl.ds(j * LANES, LANES)] = tree_sum(j)
        return 0
    lax.fori_loop(0, SLICE // LANES, s_body, 0)

    @pl.when(cid == 0)
    def _():
        pltpu.sync_copy(slice_v, s0_hbm.at[pl.ds(base_n, SLICE)])

    @pl.when(cid == 1)
    def _():
        pltpu.sync_copy(slice_v, s1_hbm.at[pl.ds(base_n, SLICE)])


@jax.jit
def _sc_coeffs(edge_index):
    mesh = plsc.VectorSubcoreMesh(
        core_axis_name="c", subcore_axis_name="s", num_cores=NUM_CORES)
    return pl.kernel(
        _sc_body,
        out_type=(
            jax.ShapeDtypeStruct((NP,), jnp.float32),
            jax.ShapeDtypeStruct((NP,), jnp.float32),
            jax.ShapeDtypeStruct((NP,), jnp.float32),
        ),
        mesh=mesh,
        compiler_params=pltpu.CompilerParams(needs_layout_passes=False),
        scratch_types=[
            pltpu.VMEM((2, WIN), jnp.int32),
            pltpu.VMEM((NP,), jnp.float32),
            pltpu.VMEM((NP,), jnp.float32),
            pltpu.VMEM((NUM_TILES, SLICE), jnp.float32),
            pltpu.VMEM((SLICE,), jnp.float32),
            pltpu.SemaphoreType.DMA,
            pltpu.SemaphoreType.DMA,
            pltpu.VMEM_SHARED((NUM_TILES, NP), jnp.float32),
            pltpu.VMEM_SHARED((NP,), jnp.float32),
        ],
    )(edge_index)


def _tc_body(x_ref, s0_ref, s1_ref, nrm_ref, w1_ref, b1_ref, w2_ref, b2_ref,
             o_ref):
    hi = jax.lax.Precision.HIGHEST
    nrm = nrm_ref[...]
    c = nrm * (s0_ref[...] + s1_ref[...]) + nrm * nrm
    c_row = c[:N].reshape(1, N)
    v = lax.dot_general(c_row, x_ref[...], (((1,), (0,)), ((), ())),
                        precision=hi, preferred_element_type=jnp.float32)
    pooled = lax.dot_general(v, w1_ref[...], (((1,), (0,)), ((), ())),
                             precision=hi, preferred_element_type=jnp.float32)
    pooled = pooled + jnp.float32(N) * b1_ref[...][None, :]
    logits = lax.dot_general(pooled, w2_ref[...], (((1,), (0,)), ((), ())),
                             precision=hi, preferred_element_type=jnp.float32)
    logits = logits + b2_ref[...][None, :]
    m = jnp.max(logits, axis=1, keepdims=True)
    e = jnp.exp(logits - m)
    p = e / jnp.sum(e, axis=1, keepdims=True)
    o_ref[...] = p[0]


@jax.jit
def _tc_head(x, s0, s1, nrm, w1, b1, w2, b2):
    return pl.pallas_call(
        _tc_body,
        out_shape=jax.ShapeDtypeStruct((L,), jnp.float32),
    )(x, s0, s1, nrm, w1, b1, w2, b2)


def kernel(x, edge_index, W1, b1, W2, b2):
    s0, s1, nrm = _sc_coeffs(edge_index)
    return _tc_head(x, s0, s1, nrm, W1, b1, W2, b2)


# speedup vs baseline: 170.0354x; 1.0041x over previous
import jax
import jax.numpy as jnp
from jax import lax
from jax.experimental import pallas as pl
from jax.experimental.pallas import tpu as pltpu
from jax.experimental.pallas import tpu_sc as plsc

N = 10000
E = 320000
D = 128
H = 32
L = 10

NUM_TILES = 16
NUM_CORES = 2
CHUNK = E // NUM_TILES
HALF = CHUNK // NUM_CORES
NP = 10240
SLICE = NP // NUM_TILES
LANES = 16
ZU = 8
EU = 5
WIN = CHUNK + 96
W1 = 10112
NGRP = CHUNK // LANES
G1 = 620


def _rsqrt16(d):
    i = lax.bitcast_convert_type(d, jnp.int32)
    i = jnp.int32(0x5F3759DF) - lax.shift_right_logical(i, 1)
    y = lax.bitcast_convert_type(i, jnp.float32)
    for _ in range(3):
        y = y * (1.5 - 0.5 * d * y * y)
    return y


def _sc_body(edge_hbm, out_hbm,
             edge_v, hist_v, norm_v, tmp_v, slice_v, sem, sem2,
             shared_h, shared_n):
    sid = lax.axis_index("s")
    cid = lax.axis_index("c")
    base_e = sid * CHUNK
    base_n = sid * SLICE
    zeros = jnp.zeros((LANES,), jnp.float32)
    ones = jnp.full((LANES,), 1.0, jnp.float32)

    start_a = base_e - lax.rem(base_e, 128)
    start_a = pl.multiple_of(start_a, 128)
    off = base_e - start_a
    cp_e1 = pltpu.async_copy(
        edge_hbm.at[:, pl.ds(start_a, W1)], edge_v.at[:, pl.ds(0, W1)], sem)
    cp_e2 = pltpu.async_copy(
        edge_hbm.at[:, pl.ds(start_a + W1, WIN - W1)],
        edge_v.at[:, pl.ds(W1, WIN - W1)], sem2)

    def zero_body(i, _):
        for u in range(ZU):
            hist_v[pl.ds((i * ZU + u) * LANES, LANES)] = zeros
        return 0

    def stage_and_reduce():
        pltpu.sync_copy(hist_v, shared_h.at[sid])
        plsc.subcore_barrier()
        pltpu.sync_copy(shared_h.at[:, pl.ds(base_n, SLICE)], tmp_v)

    lax.fori_loop(0, NP // (LANES * ZU), zero_body, 0)
    cp_e1.wait()

    def make_hist_body(g0, clamp):
        def body(i, carry):
            a, b = carry
            for u in range(EU):
                g = g0 + i * EU + u
                o_pf = off + jnp.minimum((g + 2) * LANES, clamp)
                nxt = edge_v[1, pl.ds(o_pf, LANES)]
                plsc.addupdate_scatter(hist_v, [a], ones)
                a, b = b, nxt
            return (a, b)
        return body

    lax.fori_loop(0, G1 // EU, make_hist_body(0, (G1 - 1) * LANES),
                  (edge_v[1, pl.ds(off, LANES)],
                   edge_v[1, pl.ds(off + LANES, LANES)]))
    cp_e2.wait()
    lax.fori_loop(0, (NGRP - G1) // EU,
                  make_hist_body(G1, CHUNK - LANES),
                  (edge_v[1, pl.ds(off + G1 * LANES, LANES)],
                   edge_v[1, pl.ds(off + (G1 + 1) * LANES, LANES)]))

    stage_and_reduce()

    def tree_sum(j):
        vals = [tmp_v[t, pl.ds(j * LANES, LANES)] for t in range(NUM_TILES)]
        while len(vals) > 1:
            vals = [vals[k] + vals[k + 1] for k in range(0, len(vals), 2)]
        return vals[0]

    def norm_body(j, _):
        slice_v[pl.ds(j * LANES, LANES)] = _rsqrt16(tree_sum(j) + 1.0)
        return 0
    lax.fori_loop(0, SLICE // LANES, norm_body, 0)

    pltpu.sync_copy(slice_v, shared_n.at[pl.ds(base_n, SLICE)])
    plsc.subcore_barrier()
    pltpu.sync_copy(shared_n, norm_v)

    @pl.when(cid == 0)
    def _():
        pltpu.sync_copy(slice_v, out_hbm.at[pl.ds(2 * NP + base_n, SLICE)])

    lax.fori_loop(0, NP // (LANES * ZU), zero_body, 0)
    half_off = off + cid * HALF

    def seg_body(i, carry):
        sa, va, sb, db = carry
        for u in range(EU):
            g = i * EU + u
            o_pf = half_off + jnp.minimum((g + 2) * LANES, HALF - LANES)
            sn = edge_v[0, pl.ds(o_pf, LANES)]
            dn = edge_v[1, pl.ds(o_pf, LANES)]
            vb = plsc.load_gather(norm_v, [db])
            plsc.addupdate_scatter(hist_v, [sa], va)
            sa, va, sb, db = sb, vb, sn, dn
        return (sa, va, sb, db)
    d0 = edge_v[1, pl.ds(half_off, LANES)]
    lax.fori_loop(0, HALF // (LANES * EU), seg_body,
                  (edge_v[0, pl.ds(half_off, LANES)],
                   plsc.load_gather(norm_v, [d0]),
                   edge_v[0, pl.ds(half_off + LANES, LANES)],
                   edge_v[1, pl.ds(half_off + LANES, LANES)]))

    stage_and_reduce()

    def s_body(j, _):
        slice_v[pl.ds(j * LANES, LANES)] = tree_sum(j)
        return 0
    lax.fori_loop(0, SLICE // LANES, s_body, 0)

    @pl.when(cid == 0)
    def _():
        pltpu.sync_copy(slice_v, out_hbm.at[pl.ds(base_n, SLICE)])

    @pl.when(cid == 1)
    def _():
        pltpu.sync_copy(slice_v, out_hbm.at[pl.ds(NP + base_n, SLICE)])


@jax.jit
def _sc_coeffs(edge_index):
    mesh = plsc.VectorSubcoreMesh(
        core_axis_name="c", subcore_axis_name="s", num_cores=NUM_CORES)
    return pl.kernel(
        _sc_body,
        out_type=jax.ShapeDtypeStruct((3 * NP,), jnp.float32),
        mesh=mesh,
        compiler_params=pltpu.CompilerParams(needs_layout_passes=False),
        scratch_types=[
            pltpu.VMEM((2, WIN), jnp.int32),
            pltpu.VMEM((NP,), jnp.float32),
            pltpu.VMEM((NP,), jnp.float32),
            pltpu.VMEM((NUM_TILES, SLICE), jnp.float32),
            pltpu.VMEM((SLICE,), jnp.float32),
            pltpu.SemaphoreType.DMA,
            pltpu.SemaphoreType.DMA,
            pltpu.VMEM_SHARED((NUM_TILES, NP), jnp.float32),
            pltpu.VMEM_SHARED((NP,), jnp.float32),
        ],
    )(edge_index)


def _tc_body(x_ref, s_ref, w1_ref, b1_ref, w2_ref, b2_ref, o_ref):
    hi = jax.lax.Precision.HIGHEST
    nrm = s_ref[pl.ds(2 * NP, NP)]
    c = nrm * (s_ref[pl.ds(0, NP)] + s_ref[pl.ds(NP, NP)]) + nrm * nrm
    c_row = c[:N].reshape(1, N)
    v = lax.dot_general(c_row, x_ref[...], (((1,), (0,)), ((), ())),
                        precision=hi, preferred_element_type=jnp.float32)
    pooled = lax.dot_general(v, w1_ref[...], (((1,), (0,)), ((), ())),
                             precision=hi, preferred_element_type=jnp.float32)
    pooled = pooled + jnp.float32(N) * b1_ref[...][None, :]
    logits = lax.dot_general(pooled, w2_ref[...], (((1,), (0,)), ((), ())),
                             precision=hi, preferred_element_type=jnp.float32)
    logits = logits + b2_ref[...][None, :]
    m = jnp.max(logits, axis=1, keepdims=True)
    e = jnp.exp(logits - m)
    p = e / jnp.sum(e, axis=1, keepdims=True)
    o_ref[...] = p[0]


@jax.jit
def _tc_head(x, s_all, w1, b1, w2, b2):
    return pl.pallas_call(
        _tc_body,
        out_shape=jax.ShapeDtypeStruct((L,), jnp.float32),
    )(x, s_all, w1, b1, w2, b2)


def kernel(x, edge_index, W1, b1, W2, b2):
    s_all = _sc_coeffs(edge_index)
    return _tc_head(x, s_all, W1, b1, W2, b2)
